# speedup vs baseline: 20.2282x; 20.2282x over previous
import functools

import jax
import jax.numpy as jnp
from jax import lax
from jax.experimental import pallas as pl
from jax.experimental.pallas import tpu as pltpu
from jax.experimental.pallas import tpu_sc as plsc

_V = 1000000
_D = 128
_B = 4096
_L = 50

_NC = 2
_NS = 16
_NW = _NC * _NS
_BPW = _B // _NW
_RPC = 2
_IPC = _RPC * _L
_STRIDE = 104
_CHUNKS = _BPW // _RPC
_LANES = 16
_KV = _D // _LANES


def _pool_body(idx_hbm, table_hbm, pooled_hbm, idx_v, buf, pooled_v, sem):
    wid = lax.axis_index("s") * _NC + lax.axis_index("c")
    base = wid * (_CHUNKS * _STRIDE)
    pltpu.sync_copy(idx_hbm.at[pl.ds(base, _CHUNKS * _STRIDE)], idx_v)

    def chunk_body(c, carry):
        idx_slice = idx_v.at[pl.ds(c * _STRIDE, _IPC)]
        pltpu.async_copy(table_hbm.at[idx_slice], buf, sem).wait()
        for rr in range(_RPC):
            acc = [buf[rr * _L, pl.ds(k * _LANES, _LANES)] for k in range(_KV)]
            for r in range(1, _L):
                for k in range(_KV):
                    acc[k] = acc[k] + buf[rr * _L + r, pl.ds(k * _LANES, _LANES)]
            row = c * _RPC + rr
            for k in range(_KV):
                pooled_v[row, pl.ds(k * _LANES, _LANES)] = acc[k]
        return carry

    lax.fori_loop(0, _CHUNKS, chunk_body, 0)
    pltpu.sync_copy(pooled_v, pooled_hbm.at[pl.ds(wid * _BPW, _BPW)])


_pool = functools.partial(
    pl.kernel,
    mesh=plsc.VectorSubcoreMesh(core_axis_name="c", subcore_axis_name="s"),
    out_type=jax.ShapeDtypeStruct((_B, _D), jnp.float32),
    scratch_types=[
        pltpu.VMEM((_CHUNKS * _STRIDE,), jnp.int32),
        pltpu.VMEM((_IPC, _D), jnp.float32),
        pltpu.VMEM((_BPW, _D), jnp.float32),
        pltpu.SemaphoreType.DMA,
    ],
)(_pool_body)


def _linear_body(pooled_ref, wt_ref, bias_ref, out_ref):
    out_ref[...] = (
        jnp.dot(pooled_ref[...], wt_ref[...], preferred_element_type=jnp.float32)
        + bias_ref[...]
    )


def kernel(s, table, W, b):
    s32 = s.astype(jnp.int32)
    grouped = s32.reshape(_B // _RPC, _IPC)
    padded = jnp.pad(grouped, ((0, 0), (0, _STRIDE - _IPC)))
    flat_idx = padded.reshape(-1)
    pooled = _pool(flat_idx, table)
    out = pl.pallas_call(
        _linear_body,
        out_shape=jax.ShapeDtypeStruct((_B, 2), jnp.float32),
    )(pooled, W.T.astype(jnp.float32), (_L * b).reshape(1, 2).astype(jnp.float32))
    return out


# speedup vs baseline: 22.8778x; 1.1310x over previous
import functools

import jax
import jax.numpy as jnp
from jax import lax
from jax.experimental import pallas as pl
from jax.experimental.pallas import tpu as pltpu
from jax.experimental.pallas import tpu_sc as plsc

_V = 1000000
_D = 128
_B = 4096
_L = 50

_NC = 2
_NS = 16
_NW = _NC * _NS
_BPW = _B // _NW
_RPC = 2
_IPC = _RPC * _L
_STRIDE = 104
_CHUNKS = _BPW // _RPC
_LANES = 16
_KV = _D // _LANES


def _pool_body(idx_hbm, table_hbm, pooled_hbm, idx_v, buf0, buf1, pooled_v,
               sem0, sem1):
    wid = lax.axis_index("s") * _NC + lax.axis_index("c")
    base = wid * (_CHUNKS * _STRIDE)
    pltpu.sync_copy(idx_hbm.at[pl.ds(base, _CHUNKS * _STRIDE)], idx_v)

    bufs = (buf0, buf1)
    sems = (sem0, sem1)

    def _gather(c, buf, sem):
        idx_slice = idx_v.at[pl.ds(c * _STRIDE, _IPC)]
        return pltpu.async_copy(table_hbm.at[idx_slice], buf, sem)

    _gather(0, buf0, sem0)
    _gather(1, buf1, sem1)

    def _accumulate(c, buf):
        for rr in range(_RPC):
            acc = [buf[rr * _L, pl.ds(k * _LANES, _LANES)] for k in range(_KV)]
            for r in range(1, _L):
                for k in range(_KV):
                    acc[k] = acc[k] + buf[rr * _L + r, pl.ds(k * _LANES, _LANES)]
            row = c * _RPC + rr
            for k in range(_KV):
                pooled_v[row, pl.ds(k * _LANES, _LANES)] = acc[k]

    def pair_body(c2, carry):
        for p in range(2):
            c = c2 * 2 + p
            buf, sem = bufs[p], sems[p]
            pltpu.make_async_copy(
                table_hbm.at[idx_v.at[pl.ds(c * _STRIDE, _IPC)]], buf, sem
            ).wait()
            _accumulate(c, buf)

            @pl.when(c2 < _CHUNKS // 2 - 1)
            def _():
                _gather(c + 2, buf, sem)

        return carry

    lax.fori_loop(0, _CHUNKS // 2, pair_body, 0)
    pltpu.sync_copy(pooled_v, pooled_hbm.at[pl.ds(wid * _BPW, _BPW)])


_pool = functools.partial(
    pl.kernel,
    mesh=plsc.VectorSubcoreMesh(core_axis_name="c", subcore_axis_name="s"),
    out_type=jax.ShapeDtypeStruct((_B, _D), jnp.float32),
    scratch_types=[
        pltpu.VMEM((_CHUNKS * _STRIDE,), jnp.int32),
        pltpu.VMEM((_IPC, _D), jnp.float32),
        pltpu.VMEM((_IPC, _D), jnp.float32),
        pltpu.VMEM((_BPW, _D), jnp.float32),
        pltpu.SemaphoreType.DMA,
        pltpu.SemaphoreType.DMA,
    ],
)(_pool_body)


def _linear_body(pooled_ref, wt_ref, bias_ref, out_ref):
    out_ref[...] = (
        jnp.dot(pooled_ref[...], wt_ref[...], preferred_element_type=jnp.float32)
        + bias_ref[...]
    )


def kernel(s, table, W, b):
    s32 = s.astype(jnp.int32)
    grouped = s32.reshape(_B // _RPC, _IPC)
    padded = jnp.pad(grouped, ((0, 0), (0, _STRIDE - _IPC)))
    flat_idx = padded.reshape(-1)
    pooled = _pool(flat_idx, table)
    out = pl.pallas_call(
        _linear_body,
        out_shape=jax.ShapeDtypeStruct((_B, 2), jnp.float32),
    )(pooled, W.T.astype(jnp.float32), (_L * b).reshape(1, 2).astype(jnp.float32))
    return out


# speedup vs baseline: 36.1898x; 1.5819x over previous
import functools

import jax
import jax.numpy as jnp
import numpy as np
from jax import lax
from jax.experimental import pallas as pl
from jax.experimental.pallas import tpu as pltpu
from jax.experimental.pallas import tpu_sc as plsc

_V = 1000000
_D = 128
_B = 4096
_L = 50

_NC = 2
_NS = 16
_NW = _NC * _NS
_BPW = _B // _NW
_RPC = 2
_IPC = _RPC * _L
_STRIDE = 104
_CHUNKS = _BPW // _RPC
_LANES = 16
_KV = _D // _LANES


def _pool_body(idx_hbm, rid_hbm, zero_hbm, table_hbm, pooled_hbm,
               idx_v, rid_v, buf0, buf1, pooled_v, shared_acc,
               sem_g0, sem_g1):
    cid = lax.axis_index("c")
    sid = lax.axis_index("s")
    wid = sid * _NC + cid
    base = wid * (_CHUNKS * _STRIDE)
    pltpu.sync_copy(idx_hbm.at[pl.ds(base, _CHUNKS * _STRIDE)], idx_v)
    pltpu.sync_copy(rid_hbm.at[sid], rid_v)
    pltpu.sync_copy(zero_hbm, pooled_v)
    pltpu.sync_copy(pooled_v, shared_acc.at[pl.ds(sid * _BPW, _BPW)])

    bufs = (buf0, buf1)
    gsems = (sem_g0, sem_g1)

    def _gather(c, buf, sem):
        idx_slice = idx_v.at[pl.ds(c * _STRIDE, _IPC)]
        return pltpu.async_copy(table_hbm.at[idx_slice], buf, sem)

    _gather(0, buf0, sem_g0)
    _gather(1, buf1, sem_g1)

    def pair_body(c2, carry):
        for p in range(2):
            c = c2 * 2 + p
            buf, gsem = bufs[p], gsems[p]
            pltpu.make_async_copy(
                table_hbm.at[idx_v.at[pl.ds(c * _STRIDE, _IPC)]], buf, gsem
            ).wait()
            pltpu.sync_copy(buf, shared_acc.at[rid_v.at[c]], add=True)

            @pl.when(c2 < _CHUNKS // 2 - 1)
            def _():
                _gather(c + 2, buf, gsem)

        return carry

    lax.fori_loop(0, _CHUNKS // 2, pair_body, 0)
    pltpu.sync_copy(shared_acc.at[pl.ds(sid * _BPW, _BPW)], pooled_v)
    pltpu.sync_copy(pooled_v, pooled_hbm.at[pl.ds(wid * _BPW, _BPW)])


_pool = functools.partial(
    pl.kernel,
    mesh=plsc.VectorSubcoreMesh(core_axis_name="c", subcore_axis_name="s"),
    out_type=jax.ShapeDtypeStruct((_B, _D), jnp.float32),
    scratch_types=[
        pltpu.VMEM((_CHUNKS * _STRIDE,), jnp.int32),
        pltpu.VMEM((_CHUNKS, _IPC), jnp.int32),
        pltpu.VMEM((_IPC, _D), jnp.float32),
        pltpu.VMEM((_IPC, _D), jnp.float32),
        pltpu.VMEM((_BPW, _D), jnp.float32),
        pltpu.VMEM_SHARED((_NS * _BPW, _D), jnp.float32),
        pltpu.SemaphoreType.DMA,
        pltpu.SemaphoreType.DMA,
    ],
)(_pool_body)

_RID = (
    np.repeat(np.arange(_BPW, dtype=np.int32).reshape(_CHUNKS, _RPC), _L, axis=1)
    [None, :, :]
    + (np.arange(_NS, dtype=np.int32) * _BPW)[:, None, None]
)


def _linear_body(pooled_ref, wt_ref, bias_ref, out_ref):
    out_ref[...] = (
        jnp.dot(pooled_ref[...], wt_ref[...], preferred_element_type=jnp.float32)
        + bias_ref[...]
    )


def kernel(s, table, W, b):
    s32 = s.astype(jnp.int32)
    grouped = s32.reshape(_B // _RPC, _IPC)
    padded = jnp.pad(grouped, ((0, 0), (0, _STRIDE - _IPC)))
    flat_idx = padded.reshape(-1)
    rid = jnp.asarray(_RID)
    zeros = jnp.zeros((_BPW, _D), jnp.float32)
    pooled = _pool(flat_idx, rid, zeros, table)
    out = pl.pallas_call(
        _linear_body,
        out_shape=jax.ShapeDtypeStruct((_B, 2), jnp.float32),
    )(pooled, W.T.astype(jnp.float32), (_L * b).reshape(1, 2).astype(jnp.float32))
    return out


# speedup vs baseline: 36.5131x; 1.0089x over previous
import functools

import jax
import jax.numpy as jnp
import numpy as np
from jax import lax
from jax.experimental import pallas as pl
from jax.experimental.pallas import tpu as pltpu
from jax.experimental.pallas import tpu_sc as plsc

_V = 1000000
_D = 128
_B = 4096
_L = 50

_NC = 2
_NS = 16
_NW = _NC * _NS
_BPW = _B // _NW
_RPC = 2
_IPC = _RPC * _L
_STRIDE = 104
_CHUNKS = _BPW // _RPC
_LANES = 16
_KV = _D // _LANES


_NBUF = 4


def _pool_body(idx_hbm, rid_hbm, zero_hbm, table_hbm, pooled_hbm,
               idx_v, rid_v, bufs, pooled_v, shared_acc, gsems, ssems):
    cid = lax.axis_index("c")
    sid = lax.axis_index("s")
    wid = sid * _NC + cid
    pltpu.sync_copy(idx_hbm.at[wid], idx_v)
    pltpu.sync_copy(rid_hbm.at[sid], rid_v)
    pltpu.sync_copy(zero_hbm, pooled_v)
    pltpu.sync_copy(pooled_v, shared_acc.at[pl.ds(sid * _BPW, _BPW)])

    def _gather(c, p):
        return pltpu.async_copy(table_hbm.at[idx_v.at[c]], bufs[p], gsems[p])

    _gather(0, 0)
    _gather(1, 1)

    def quad_body(c4, carry):
        for p in range(_NBUF):
            c = c4 * _NBUF + p
            pltpu.make_async_copy(
                table_hbm.at[idx_v.at[c]], bufs[p], gsems[p]
            ).wait()
            pltpu.async_copy(bufs[p], shared_acc.at[rid_v.at[c]], ssems[p],
                             add=True)
            q = (p + 2) % _NBUF
            prev = c - (_NBUF - 2)

            @pl.when(jnp.logical_and(prev >= 0, c + 2 < _CHUNKS))
            def _():
                pltpu.make_async_copy(
                    bufs[q], shared_acc.at[rid_v.at[prev]], ssems[q]
                ).wait()

            @pl.when(c + 2 < _CHUNKS)
            def _():
                _gather(c + 2, q)

        return carry

    lax.fori_loop(0, _CHUNKS // _NBUF, quad_body, 0)
    for p in range(_NBUF):
        c = _CHUNKS - _NBUF + p
        pltpu.make_async_copy(
            bufs[p % _NBUF], shared_acc.at[rid_v.at[c]], ssems[c % _NBUF]
        ).wait()
    pltpu.sync_copy(shared_acc.at[pl.ds(sid * _BPW, _BPW)], pooled_v)
    pltpu.sync_copy(pooled_v, pooled_hbm.at[pl.ds(wid * _BPW, _BPW)])


_pool = functools.partial(
    pl.kernel,
    mesh=plsc.VectorSubcoreMesh(core_axis_name="c", subcore_axis_name="s"),
    out_type=jax.ShapeDtypeStruct((_B, _D), jnp.float32),
    scratch_types=[
        pltpu.VMEM((_CHUNKS, _IPC), jnp.int32),
        pltpu.VMEM((_CHUNKS, _IPC), jnp.int32),
        tuple(pltpu.VMEM((_IPC, _D), jnp.float32) for _ in range(_NBUF)),
        pltpu.VMEM((_BPW, _D), jnp.float32),
        pltpu.VMEM_SHARED((_NS * _BPW, _D), jnp.float32),
        tuple(pltpu.SemaphoreType.DMA for _ in range(_NBUF)),
        tuple(pltpu.SemaphoreType.DMA for _ in range(_NBUF)),
    ],
)(_pool_body)

_RID = (
    np.repeat(np.arange(_BPW, dtype=np.int32).reshape(_CHUNKS, _RPC), _L, axis=1)
    [None, :, :]
    + (np.arange(_NS, dtype=np.int32) * _BPW)[:, None, None]
)


def _linear_body(pooled_ref, wt_ref, bias_ref, out_ref):
    out_ref[...] = (
        jnp.dot(pooled_ref[...], wt_ref[...], preferred_element_type=jnp.float32)
        + bias_ref[...]
    )


def kernel(s, table, W, b):
    s32 = s.astype(jnp.int32)
    flat_idx = s32.reshape(_NW, _CHUNKS, _IPC)
    rid = jnp.asarray(_RID)
    zeros = jnp.zeros((_BPW, _D), jnp.float32)
    pooled = _pool(flat_idx, rid, zeros, table)
    out = pl.pallas_call(
        _linear_body,
        out_shape=jax.ShapeDtypeStruct((_B, 2), jnp.float32),
    )(pooled, W.T.astype(jnp.float32), (_L * b).reshape(1, 2).astype(jnp.float32))
    return out
